# Initial kernel scaffold; baseline (speedup 1.0000x reference)
import jax
import jax.numpy as jnp
from jax.experimental import pallas as pl


def kernel(classifications, regressions, anchors, annotations, ignores):
    raise NotImplementedError("write your pallas kernel here")


# speedup vs baseline: 1.2667x; 1.2667x over previous
import functools

import jax
import jax.numpy as jnp
from jax.experimental import pallas as pl
from jax.experimental.pallas import tpu as pltpu

ALPHA = 0.25
NB = 2000


def _focal_block(cls_ref, reg_ref, anc_ref, ann_ref, ign_ref, out_ref):
    nb = pl.program_id(1)

    ax0 = anc_ref[0, :, 0:1]
    ay0 = anc_ref[0, :, 1:2]
    ax1 = anc_ref[0, :, 2:3]
    ay1 = anc_ref[0, :, 3:4]
    aw = ax1 - ax0
    ah = ay1 - ay0
    acx = ax0 + 0.5 * aw
    acy = ay0 + 0.5 * ah
    area_a = aw * ah

    bx0 = ann_ref[0, 0:1, :]
    by0 = ann_ref[0, 1:2, :]
    bx1 = ann_ref[0, 2:3, :]
    by1 = ann_ref[0, 3:4, :]
    blab = ann_ref[0, 4:5, :]
    iw = jnp.maximum(jnp.minimum(ax1, bx1) - jnp.maximum(ax0, bx0), 0.0)
    ih = jnp.maximum(jnp.minimum(ay1, by1) - jnp.maximum(ay0, by0), 0.0)
    inter = iw * ih
    ua = jnp.maximum(area_a + (bx1 - bx0) * (by1 - by0) - inter, 1e-8)
    iou = inter / ua
    iou_max = jnp.max(iou, axis=1, keepdims=True)
    m_iota = jax.lax.broadcasted_iota(jnp.int32, iou.shape, 1)
    m_count = iou.shape[1]
    argmax = jnp.min(jnp.where(iou == iou_max, m_iota, m_count), axis=1,
                     keepdims=True)
    onehot_m = (m_iota == argmax)

    gx0 = ign_ref[0, 0:1, :]
    gy0 = ign_ref[0, 1:2, :]
    gx1 = ign_ref[0, 2:3, :]
    gy1 = ign_ref[0, 3:4, :]
    giw = jnp.maximum(jnp.minimum(ax1, gx1) - jnp.maximum(ax0, gx0), 0.0)
    gih = jnp.maximum(jnp.minimum(ay1, gy1) - jnp.maximum(ay0, gy0), 0.0)
    ginter = giw * gih
    gua = jnp.maximum(area_a + (gx1 - gx0) * (gy1 - gy0) - ginter, 1e-8)
    keep = jnp.max(ginter / gua, axis=1, keepdims=True) < 0.5

    pos = (iou_max >= 0.5) & keep
    neg = (iou_max < 0.4) & keep
    posf = pos.astype(jnp.float32)
    num_pos = jnp.sum(posf)

    def pick(row):
        return jnp.sum(jnp.where(onehot_m, row, 0.0), axis=1, keepdims=True)

    gx0a = pick(bx0)
    gy0a = pick(by0)
    gx1a = pick(bx1)
    gy1a = pick(by1)
    labels = pick(blab).astype(jnp.int32)

    x = jnp.clip(cls_ref[0], 1e-4, 1.0 - 1e-4)
    t0 = x * x * (-jnp.log(1.0 - x))
    row_sum = jnp.sum(t0, axis=1, keepdims=True)
    c_iota = jax.lax.broadcasted_iota(jnp.int32, x.shape, 1)
    x_l = jnp.sum(jnp.where(c_iota == labels, x, 0.0), axis=1, keepdims=True)

    t0_l = x_l * x_l * (-jnp.log(1.0 - x_l))
    t1_l = ALPHA * (1.0 - x_l) * (1.0 - x_l) * (-jnp.log(x_l))
    row_loss = jnp.where(
        pos, (1.0 - ALPHA) * (row_sum - t0_l) + t1_l,
        jnp.where(neg, (1.0 - ALPHA) * row_sum, 0.0))
    cls_sum = jnp.sum(row_loss)

    gw_raw = gx1a - gx0a
    gh_raw = gy1a - gy0a
    gcx = gx0a + 0.5 * gw_raw
    gcy = gy0a + 0.5 * gh_raw
    gw = jnp.maximum(gw_raw, 1.0)
    gh = jnp.maximum(gh_raw, 1.0)
    t_0 = ((gcx - acx) / aw) / 0.1
    t_1 = ((gcy - acy) / ah) / 0.1
    t_2 = jnp.log(gw / aw) / 0.2
    t_3 = jnp.log(gh / ah) / 0.2

    def smooth_l1(t, r):
        d = jnp.abs(t - r)
        return jnp.where(d <= 1.0 / 9.0, 0.5 * 9.0 * d * d, d - 0.5 / 9.0)

    rl = (smooth_l1(t_0, reg_ref[0, :, 0:1]) +
          smooth_l1(t_1, reg_ref[0, :, 1:2]) +
          smooth_l1(t_2, reg_ref[0, :, 2:3]) +
          smooth_l1(t_3, reg_ref[0, :, 3:4]))
    reg_sum = jnp.sum(rl * posf)

    s_iota = jax.lax.broadcasted_iota(jnp.int32, (8, 128), 0)
    l_iota = jax.lax.broadcasted_iota(jnp.int32, (8, 128), 1)
    lane0 = l_iota == 0
    vec = (jnp.where(lane0 & (s_iota == 0), cls_sum, 0.0) +
           jnp.where(lane0 & (s_iota == 1), reg_sum, 0.0) +
           jnp.where(lane0 & (s_iota == 2), num_pos, 0.0))

    @pl.when(nb == 0)
    def _():
        out_ref[0] = jnp.zeros((8, 128), jnp.float32)

    out_ref[0] += vec


@jax.jit
def kernel(classifications, regressions, anchors, annotations, ignores):
    B, N, C = classifications.shape
    ann_t = jnp.transpose(annotations, (0, 2, 1))
    ign_t = jnp.transpose(ignores, (0, 2, 1))
    nblk = N // NB

    out = pl.pallas_call(
        _focal_block,
        grid=(B, nblk),
        in_specs=[
            pl.BlockSpec((1, NB, C), lambda j, nb: (j, nb, 0)),
            pl.BlockSpec((1, NB, 4), lambda j, nb: (j, nb, 0)),
            pl.BlockSpec((1, NB, 4), lambda j, nb: (0, nb, 0)),
            pl.BlockSpec((1, 5, annotations.shape[1]), lambda j, nb: (j, 0, 0)),
            pl.BlockSpec((1, 5, ignores.shape[1]), lambda j, nb: (j, 0, 0)),
        ],
        out_specs=pl.BlockSpec((1, 8, 128), lambda j, nb: (j, 0, 0)),
        out_shape=jax.ShapeDtypeStruct((B, 8, 128), jnp.float32),
        compiler_params=pltpu.CompilerParams(
            dimension_semantics=("parallel", "arbitrary")),
    )(classifications, regressions, anchors, ann_t, ign_t)

    cls_sums = out[:, 0, 0]
    reg_sums = out[:, 1, 0]
    npos = out[:, 2, 0]
    cls_losses = cls_sums / jnp.maximum(npos, 1.0)
    reg_losses = reg_sums / jnp.maximum(npos * 4.0, 1.0)
    return jnp.stack([jnp.mean(cls_losses), jnp.mean(reg_losses)])


# speedup vs baseline: 7.9746x; 6.2956x over previous
import jax
import jax.numpy as jnp
from jax.experimental import pallas as pl
from jax.experimental.pallas import tpu as pltpu

ALPHA = 0.25


def _focal_block(cls_ref, reg_ref, anc_ref, ann_ref, ign_ref, out_ref):
    ax0 = anc_ref[0, 0:1, :]
    ay0 = anc_ref[0, 1:2, :]
    ax1 = anc_ref[0, 2:3, :]
    ay1 = anc_ref[0, 3:4, :]
    aw = ax1 - ax0
    ah = ay1 - ay0
    acx = ax0 + 0.5 * aw
    acy = ay0 + 0.5 * ah
    area_a = aw * ah

    ann = ann_ref[0]
    bx0 = ann[:, 0:1]
    by0 = ann[:, 1:2]
    bx1 = ann[:, 2:3]
    by1 = ann[:, 3:4]
    blab = ann[:, 4:5]
    iw = jnp.maximum(jnp.minimum(ax1, bx1) - jnp.maximum(ax0, bx0), 0.0)
    ih = jnp.maximum(jnp.minimum(ay1, by1) - jnp.maximum(ay0, by0), 0.0)
    inter = iw * ih
    ua = jnp.maximum(area_a + (bx1 - bx0) * (by1 - by0) - inter, 1e-8)
    iou = inter / ua
    iou_max = jnp.max(iou, axis=0, keepdims=True)
    m_iota = jax.lax.broadcasted_iota(jnp.int32, iou.shape, 0)
    m_count = iou.shape[0]
    argmax = jnp.min(jnp.where(iou == iou_max, m_iota, m_count), axis=0,
                     keepdims=True)
    onehot_m = (m_iota == argmax)

    ign = ign_ref[0]
    gx0 = ign[:, 0:1]
    gy0 = ign[:, 1:2]
    gx1 = ign[:, 2:3]
    gy1 = ign[:, 3:4]
    giw = jnp.maximum(jnp.minimum(ax1, gx1) - jnp.maximum(ax0, gx0), 0.0)
    gih = jnp.maximum(jnp.minimum(ay1, gy1) - jnp.maximum(ay0, gy0), 0.0)
    ginter = giw * gih
    gua = jnp.maximum(area_a + (gx1 - gx0) * (gy1 - gy0) - ginter, 1e-8)
    keep = jnp.max(ginter / gua, axis=0, keepdims=True) < 0.5

    pos = (iou_max >= 0.5) & keep
    neg = (iou_max < 0.4) & keep
    posf = pos.astype(jnp.float32)
    num_pos = jnp.sum(posf)

    def pick(col):
        return jnp.sum(jnp.where(onehot_m, col, 0.0), axis=0, keepdims=True)

    gx0a = pick(bx0)
    gy0a = pick(by0)
    gx1a = pick(bx1)
    gy1a = pick(by1)
    labels = pick(blab).astype(jnp.int32)

    x = jnp.clip(cls_ref[0], 1e-4, 1.0 - 1e-4)
    t0 = x * x * (-jnp.log(1.0 - x))
    col_sum = jnp.sum(t0, axis=0, keepdims=True)
    c_iota = jax.lax.broadcasted_iota(jnp.int32, x.shape, 0)
    x_l = jnp.sum(jnp.where(c_iota == labels, x, 0.0), axis=0, keepdims=True)

    t0_l = x_l * x_l * (-jnp.log(1.0 - x_l))
    t1_l = ALPHA * (1.0 - x_l) * (1.0 - x_l) * (-jnp.log(x_l))
    row_loss = jnp.where(
        pos, (1.0 - ALPHA) * (col_sum - t0_l) + t1_l,
        jnp.where(neg, (1.0 - ALPHA) * col_sum, 0.0))
    cls_sum = jnp.sum(row_loss)

    gw_raw = gx1a - gx0a
    gh_raw = gy1a - gy0a
    gcx = gx0a + 0.5 * gw_raw
    gcy = gy0a + 0.5 * gh_raw
    gw = jnp.maximum(gw_raw, 1.0)
    gh = jnp.maximum(gh_raw, 1.0)
    t_0 = ((gcx - acx) / aw) / 0.1
    t_1 = ((gcy - acy) / ah) / 0.1
    t_2 = jnp.log(gw / aw) / 0.2
    t_3 = jnp.log(gh / ah) / 0.2

    def smooth_l1(t, r):
        d = jnp.abs(t - r)
        return jnp.where(d <= 1.0 / 9.0, 0.5 * 9.0 * d * d, d - 0.5 / 9.0)

    reg = reg_ref[0]
    rl = (smooth_l1(t_0, reg[0:1, :]) +
          smooth_l1(t_1, reg[1:2, :]) +
          smooth_l1(t_2, reg[2:3, :]) +
          smooth_l1(t_3, reg[3:4, :]))
    reg_sum = jnp.sum(rl * posf)

    s_iota = jax.lax.broadcasted_iota(jnp.int32, (8, 128), 0)
    l_iota = jax.lax.broadcasted_iota(jnp.int32, (8, 128), 1)
    lane0 = l_iota == 0
    vec = (jnp.where(lane0 & (s_iota == 0), cls_sum, 0.0) +
           jnp.where(lane0 & (s_iota == 1), reg_sum, 0.0) +
           jnp.where(lane0 & (s_iota == 2), num_pos, 0.0))
    out_ref[0] = vec


@jax.jit
def kernel(classifications, regressions, anchors, annotations, ignores):
    B, N, C = classifications.shape
    M = annotations.shape[1]
    G = ignores.shape[1]
    cls_t = jnp.transpose(classifications, (0, 2, 1))
    reg_t = jnp.transpose(regressions, (0, 2, 1))
    anc_t = jnp.transpose(anchors, (0, 2, 1))

    out = pl.pallas_call(
        _focal_block,
        grid=(B,),
        in_specs=[
            pl.BlockSpec((1, C, N), lambda j: (j, 0, 0)),
            pl.BlockSpec((1, 4, N), lambda j: (j, 0, 0)),
            pl.BlockSpec((1, 4, N), lambda j: (0, 0, 0)),
            pl.BlockSpec((1, M, 5), lambda j: (j, 0, 0)),
            pl.BlockSpec((1, G, 5), lambda j: (j, 0, 0)),
        ],
        out_specs=pl.BlockSpec((1, 8, 128), lambda j: (j, 0, 0)),
        out_shape=jax.ShapeDtypeStruct((B, 8, 128), jnp.float32),
        compiler_params=pltpu.CompilerParams(
            dimension_semantics=("parallel",)),
    )(cls_t, reg_t, anc_t, annotations, ignores)

    cls_sums = out[:, 0, 0]
    reg_sums = out[:, 1, 0]
    npos = out[:, 2, 0]
    cls_losses = cls_sums / jnp.maximum(npos, 1.0)
    reg_losses = reg_sums / jnp.maximum(npos * 4.0, 1.0)
    return jnp.stack([jnp.mean(cls_losses), jnp.mean(reg_losses)])
